# Initial kernel scaffold; baseline (speedup 1.0000x reference)
import jax
import jax.numpy as jnp
from jax.experimental import pallas as pl


def kernel(species_ids, move_ids, ability_ids, item_ids, tera_ids, numeric, sp_table, mv_table, ab_table, it_table, te_table, W_pe, b_pe, W1, b1, g1, be1, W2, b2, g2, be2, W3, b3, g3, be3, Wv1, bv1, Wv2, bv2, Wp1, bp1, Wp2, bp2):
    raise NotImplementedError("write your pallas kernel here")


# speedup vs baseline: 4.8644x; 4.8644x over previous
import functools

import jax
import jax.numpy as jnp
from jax import lax
from jax.experimental import pallas as pl
from jax.experimental.pallas import tpu as pltpu
from jax.experimental.pallas import tpu_sc as plsc

B = 16384
S = 4
M = 4
EMBED = 32
FEAT = 16
POKE = 48
HID = 256
NUMERIC = 24
NUM_ACTIONS = 100

NP = B * S
NC, NS = 2, 16
NW = NC * NS
CHUNK = 512
PER_W = NP // NW
N_CHUNKS = PER_W // CHUNK

R = 512


def _sc_gather(sp_ids, mv_ids, ab_ids, it_ids, te_ids,
               sp_table, mv_table, ab_table, it_table, te_table):
    mesh = plsc.VectorSubcoreMesh(core_axis_name="c", subcore_axis_name="s")

    @functools.partial(
        pl.kernel,
        mesh=mesh,
        compiler_params=pltpu.CompilerParams(use_tc_tiling_on_sc=False),
        out_type=(
            jax.ShapeDtypeStruct((NP, EMBED), jnp.float32),
            jax.ShapeDtypeStruct((NP * M, FEAT), jnp.float32),
            jax.ShapeDtypeStruct((NP, FEAT), jnp.float32),
            jax.ShapeDtypeStruct((NP, FEAT), jnp.float32),
            jax.ShapeDtypeStruct((NP, FEAT), jnp.float32),
        ),
        scratch_types=[
            pltpu.VMEM((CHUNK,), jnp.int32),
            pltpu.VMEM((CHUNK * M,), jnp.int32),
            pltpu.VMEM((CHUNK,), jnp.int32),
            pltpu.VMEM((CHUNK,), jnp.int32),
            pltpu.VMEM((CHUNK,), jnp.int32),
            pltpu.VMEM((CHUNK, EMBED), jnp.float32),
            pltpu.VMEM((CHUNK * M, FEAT), jnp.float32),
            pltpu.VMEM((CHUNK, FEAT), jnp.float32),
            pltpu.VMEM((CHUNK, FEAT), jnp.float32),
            pltpu.VMEM((CHUNK, FEAT), jnp.float32),
            pltpu.SemaphoreType.DMA,
        ],
    )
    def k(sp_ids_h, mv_ids_h, ab_ids_h, it_ids_h, te_ids_h,
          sp_t, mv_t, ab_t, it_t, te_t,
          sp_o, mv_o, ab_o, it_o, te_o,
          i_sp, i_mv, i_ab, i_it, i_te,
          d_sp, d_mv, d_ab, d_it, d_te, sem):
        wid = lax.axis_index("s") * NC + lax.axis_index("c")
        for c in range(N_CHUNKS):
            base = (wid * N_CHUNKS + c) * CHUNK
            mbase = base * M
            pltpu.sync_copy(sp_ids_h.at[pl.ds(base, CHUNK)], i_sp)
            pltpu.sync_copy(mv_ids_h.at[pl.ds(mbase, CHUNK * M)], i_mv)
            pltpu.sync_copy(ab_ids_h.at[pl.ds(base, CHUNK)], i_ab)
            pltpu.sync_copy(it_ids_h.at[pl.ds(base, CHUNK)], i_it)
            pltpu.sync_copy(te_ids_h.at[pl.ds(base, CHUNK)], i_te)
            cps = [
                pltpu.async_copy(sp_t.at[i_sp], d_sp, sem),
                pltpu.async_copy(mv_t.at[i_mv], d_mv, sem),
                pltpu.async_copy(ab_t.at[i_ab], d_ab, sem),
                pltpu.async_copy(it_t.at[i_it], d_it, sem),
                pltpu.async_copy(te_t.at[i_te], d_te, sem),
            ]
            for cp in cps:
                cp.wait()
            pltpu.sync_copy(d_sp, sp_o.at[pl.ds(base, CHUNK)])
            pltpu.sync_copy(d_mv, mv_o.at[pl.ds(mbase, CHUNK * M)])
            pltpu.sync_copy(d_ab, ab_o.at[pl.ds(base, CHUNK)])
            pltpu.sync_copy(d_it, it_o.at[pl.ds(base, CHUNK)])
            pltpu.sync_copy(d_te, te_o.at[pl.ds(base, CHUNK)])

    return k(sp_ids, mv_ids, ab_ids, it_ids, te_ids,
             sp_table, mv_table, ab_table, it_table, te_table)


def _mlp_body(sp_r, mv_r, ab_r, it_r, te_r, num_r,
              wsp_r, wmv_r, wab_r, wit_r, wte_r, bpe_r,
              w1f_r, w1n_r, b1_r, w2_r, b2_r, w3_r, b3_r,
              wv1_r, bv1_r, wv2_r, bv2_r,
              wp1x_r, wp1s_r, bp1_r, wp2_r, bp2_r,
              v_o, p0_o, p1_o):
    dot = functools.partial(jnp.dot, preferred_element_type=jnp.float32)
    enc = dot(sp_r[:], wsp_r[:]) + dot(mv_r[:], wmv_r[:]) \
        + dot(ab_r[:], wab_r[:]) + dot(it_r[:], wit_r[:]) \
        + dot(te_r[:], wte_r[:]) + bpe_r[:]
    enc = jnp.maximum(enc, 0.0)
    x = jnp.maximum(dot(enc, w1f_r[:]) + dot(num_r[:], w1n_r[:]) + b1_r[:], 0.0)
    x = jnp.maximum(dot(x, w2_r[:]) + b2_r[:], 0.0)
    x = jnp.maximum(dot(x, w3_r[:]) + b3_r[:], 0.0)
    v = dot(jnp.maximum(dot(x, wv1_r[:]) + bv1_r[:], 0.0), wv2_r[:]) + bv2_r[:]
    v_o[:] = v
    for i, p_o in ((0, p0_o), (1, p1_o)):
        slot = enc[:, i * POKE:(i + 1) * POKE]
        h = jnp.maximum(dot(x, wp1x_r[:]) + dot(slot, wp1s_r[:]) + bp1_r[:], 0.0)
        p_o[:] = dot(h, wp2_r[:]) + bp2_r[:]


def kernel(species_ids, move_ids, ability_ids, item_ids, tera_ids, numeric, sp_table, mv_table, ab_table, it_table, te_table, W_pe, b_pe, W1, b1, g1, be1, W2, b2, g2, be2, W3, b3, g3, be3, Wv1, bv1, Wv2, bv2, Wp1, bp1, Wp2, bp2):
    f32 = jnp.float32
    sp_g, mv_g, ab_g, it_g, te_g = _sc_gather(
        species_ids.reshape(-1).astype(jnp.int32),
        move_ids.reshape(-1).astype(jnp.int32),
        ability_ids.reshape(-1).astype(jnp.int32),
        item_ids.reshape(-1).astype(jnp.int32),
        tera_ids.reshape(-1).astype(jnp.int32),
        sp_table, mv_table, ab_table, it_table, te_table)
    sp2 = sp_g.reshape(B, S * EMBED)
    mv2 = mv_g.reshape(B, S * M * FEAT)
    ab2 = ab_g.reshape(B, S * FEAT)
    it2 = it_g.reshape(B, S * FEAT)
    te2 = te_g.reshape(B, S * FEAT)

    def bd(blk):
        z = jnp.zeros((S * blk.shape[0], S * POKE), f32)
        for s in range(S):
            z = z.at[s * blk.shape[0]:(s + 1) * blk.shape[0],
                     s * POKE:(s + 1) * POKE].set(blk)
        return z

    wsp = bd(W_pe[0:32])
    wmv = bd(jnp.tile(W_pe[32:48], (M, 1)))
    wab = bd(W_pe[48:64])
    wit = bd(W_pe[64:80])
    wte = bd(W_pe[80:96])
    bpe = jnp.tile(b_pe, S).reshape(1, S * POKE)

    inv = 1.0 / jnp.sqrt(1.0 + 1e-5)
    w1 = W1 * (g1 * inv)[None, :]
    b1f = (b1 * g1 * inv + be1).reshape(1, HID)
    w2 = W2 * (g2 * inv)[None, :]
    b2f = (b2 * g2 * inv + be2).reshape(1, HID)
    w3 = W3 * (g3 * inv)[None, :]
    b3f = (b3 * g3 * inv + be3).reshape(1, HID // 2)
    w1f, w1n = w1[:S * POKE], w1[S * POKE:]

    grid = (B // R,)
    row = lambda c: pl.BlockSpec((R, c), lambda i: (i, 0))
    full = lambda a: pl.BlockSpec(a.shape, lambda i: (0,) * a.ndim)
    wargs = (wsp, wmv, wab, wit, wte, bpe,
             w1f, w1n, b1f, w2, b2f, w3, b3f,
             Wv1, bv1.reshape(1, 64), Wv2, bv2.reshape(1, 1),
             Wp1[:HID // 2], Wp1[HID // 2:], bp1.reshape(1, HID // 2),
             Wp2, bp2.reshape(1, NUM_ACTIONS))
    v, p0, p1 = pl.pallas_call(
        _mlp_body,
        grid=grid,
        in_specs=[row(S * EMBED), row(S * M * FEAT), row(S * FEAT),
                  row(S * FEAT), row(S * FEAT), row(NUMERIC)]
                 + [full(a) for a in wargs],
        out_specs=[row(1), row(NUM_ACTIONS), row(NUM_ACTIONS)],
        out_shape=[jax.ShapeDtypeStruct((B, 1), f32),
                   jax.ShapeDtypeStruct((B, NUM_ACTIONS), f32),
                   jax.ShapeDtypeStruct((B, NUM_ACTIONS), f32)],
        compiler_params=pltpu.CompilerParams(
            dimension_semantics=("parallel",)),
    )(sp2, mv2, ab2, it2, te2, numeric, *wargs)
    return (v.reshape(B), p0, p1)


# speedup vs baseline: 4.9313x; 1.0138x over previous
import functools

import jax
import jax.numpy as jnp
from jax import lax
from jax.experimental import pallas as pl
from jax.experimental.pallas import tpu as pltpu
from jax.experimental.pallas import tpu_sc as plsc

B = 16384
S = 4
M = 4
EMBED = 32
FEAT = 16
POKE = 48
HID = 256
NUMERIC = 24
NUM_ACTIONS = 100

NP = B * S
NC, NS = 2, 16
NW = NC * NS
CHUNK = 256
PER_W = NP // NW
N_CHUNKS = PER_W // CHUNK
NBUF = 2

R = 512


def _sc_gather(sp_ids, mv_ids, ab_ids, it_ids, te_ids,
               sp_table, mv_table, ab_table, it_table, te_table):
    mesh = plsc.VectorSubcoreMesh(core_axis_name="c", subcore_axis_name="s")

    @functools.partial(
        pl.kernel,
        mesh=mesh,
        compiler_params=pltpu.CompilerParams(use_tc_tiling_on_sc=False),
        out_type=(
            jax.ShapeDtypeStruct((NP, EMBED), jnp.float32),
            jax.ShapeDtypeStruct((NP * M, FEAT), jnp.float32),
            jax.ShapeDtypeStruct((NP, FEAT), jnp.float32),
            jax.ShapeDtypeStruct((NP, FEAT), jnp.float32),
            jax.ShapeDtypeStruct((NP, FEAT), jnp.float32),
        ),
        scratch_types=[
            pltpu.VMEM((PER_W,), jnp.int32),
            pltpu.VMEM((PER_W * M,), jnp.int32),
            pltpu.VMEM((PER_W,), jnp.int32),
            pltpu.VMEM((PER_W,), jnp.int32),
            pltpu.VMEM((PER_W,), jnp.int32),
            pltpu.VMEM((NBUF, CHUNK, EMBED), jnp.float32),
            pltpu.VMEM((NBUF, CHUNK * M, FEAT), jnp.float32),
            pltpu.VMEM((NBUF, CHUNK, FEAT), jnp.float32),
            pltpu.VMEM((NBUF, CHUNK, FEAT), jnp.float32),
            pltpu.VMEM((NBUF, CHUNK, FEAT), jnp.float32),
            pltpu.SemaphoreType.DMA((NBUF,)),
            pltpu.SemaphoreType.DMA((NBUF,)),
            pltpu.SemaphoreType.DMA,
        ],
    )
    def k(sp_ids_h, mv_ids_h, ab_ids_h, it_ids_h, te_ids_h,
          sp_t, mv_t, ab_t, it_t, te_t,
          sp_o, mv_o, ab_o, it_o, te_o,
          i_sp, i_mv, i_ab, i_it, i_te,
          d_sp, d_mv, d_ab, d_it, d_te, gsem, wsem, isem):
        wid = lax.axis_index("s") * NC + lax.axis_index("c")
        w0 = wid * PER_W
        idcps = [
            pltpu.async_copy(sp_ids_h.at[pl.ds(w0, PER_W)], i_sp, isem),
            pltpu.async_copy(mv_ids_h.at[pl.ds(w0 * M, PER_W * M)], i_mv, isem),
            pltpu.async_copy(ab_ids_h.at[pl.ds(w0, PER_W)], i_ab, isem),
            pltpu.async_copy(it_ids_h.at[pl.ds(w0, PER_W)], i_it, isem),
            pltpu.async_copy(te_ids_h.at[pl.ds(w0, PER_W)], i_te, isem),
        ]
        for cp in idcps:
            cp.wait()

        def fire_gather(c):
            b = c % NBUF
            o = c * CHUNK
            return [
                pltpu.async_copy(sp_t.at[i_sp.at[pl.ds(o, CHUNK)]],
                                 d_sp.at[b], gsem.at[b]),
                pltpu.async_copy(mv_t.at[i_mv.at[pl.ds(o * M, CHUNK * M)]],
                                 d_mv.at[b], gsem.at[b]),
                pltpu.async_copy(ab_t.at[i_ab.at[pl.ds(o, CHUNK)]],
                                 d_ab.at[b], gsem.at[b]),
                pltpu.async_copy(it_t.at[i_it.at[pl.ds(o, CHUNK)]],
                                 d_it.at[b], gsem.at[b]),
                pltpu.async_copy(te_t.at[i_te.at[pl.ds(o, CHUNK)]],
                                 d_te.at[b], gsem.at[b]),
            ]

        def fire_write(c):
            b = c % NBUF
            base = w0 + c * CHUNK
            return [
                pltpu.async_copy(d_sp.at[b], sp_o.at[pl.ds(base, CHUNK)],
                                 wsem.at[b]),
                pltpu.async_copy(d_mv.at[b], mv_o.at[pl.ds(base * M, CHUNK * M)],
                                 wsem.at[b]),
                pltpu.async_copy(d_ab.at[b], ab_o.at[pl.ds(base, CHUNK)],
                                 wsem.at[b]),
                pltpu.async_copy(d_it.at[b], it_o.at[pl.ds(base, CHUNK)],
                                 wsem.at[b]),
                pltpu.async_copy(d_te.at[b], te_o.at[pl.ds(base, CHUNK)],
                                 wsem.at[b]),
            ]

        gcps = {c: fire_gather(c) for c in range(min(NBUF, N_CHUNKS))}
        wcps = {}
        for c in range(N_CHUNKS):
            for cp in gcps.pop(c):
                cp.wait()
            wcps[c] = fire_write(c)
            nxt = c + NBUF
            if nxt < N_CHUNKS:
                for cp in wcps.pop(c):
                    cp.wait()
                gcps[nxt] = fire_gather(nxt)
        for c, cps in wcps.items():
            for cp in cps:
                cp.wait()

    return k(sp_ids, mv_ids, ab_ids, it_ids, te_ids,
             sp_table, mv_table, ab_table, it_table, te_table)


def _mlp_body(sp_r, mv_r, ab_r, it_r, te_r, num_r,
              wsp_r, wmv_r, wab_r, wit_r, wte_r, bpe_r,
              w1f_r, w1n_r, b1_r, w2_r, b2_r, w3_r, b3_r,
              wv1_r, bv1_r, wv2_r, bv2_r,
              wp1x_r, wp1s_r, bp1_r, wp2_r, bp2_r,
              v_o, p0_o, p1_o):
    dot = functools.partial(jnp.dot, preferred_element_type=jnp.float32)
    enc = dot(sp_r[:], wsp_r[:]) + dot(mv_r[:], wmv_r[:]) \
        + dot(ab_r[:], wab_r[:]) + dot(it_r[:], wit_r[:]) \
        + dot(te_r[:], wte_r[:]) + bpe_r[:]
    enc = jnp.maximum(enc, 0.0)
    x = jnp.maximum(dot(enc, w1f_r[:]) + dot(num_r[:], w1n_r[:]) + b1_r[:], 0.0)
    x = jnp.maximum(dot(x, w2_r[:]) + b2_r[:], 0.0)
    x = jnp.maximum(dot(x, w3_r[:]) + b3_r[:], 0.0)
    v = dot(jnp.maximum(dot(x, wv1_r[:]) + bv1_r[:], 0.0), wv2_r[:]) + bv2_r[:]
    v_o[:] = v
    for i, p_o in ((0, p0_o), (1, p1_o)):
        slot = enc[:, i * POKE:(i + 1) * POKE]
        h = jnp.maximum(dot(x, wp1x_r[:]) + dot(slot, wp1s_r[:]) + bp1_r[:], 0.0)
        p_o[:] = dot(h, wp2_r[:]) + bp2_r[:]


def kernel(species_ids, move_ids, ability_ids, item_ids, tera_ids, numeric, sp_table, mv_table, ab_table, it_table, te_table, W_pe, b_pe, W1, b1, g1, be1, W2, b2, g2, be2, W3, b3, g3, be3, Wv1, bv1, Wv2, bv2, Wp1, bp1, Wp2, bp2):
    f32 = jnp.float32
    sp_g, mv_g, ab_g, it_g, te_g = _sc_gather(
        species_ids.reshape(-1).astype(jnp.int32),
        move_ids.reshape(-1).astype(jnp.int32),
        ability_ids.reshape(-1).astype(jnp.int32),
        item_ids.reshape(-1).astype(jnp.int32),
        tera_ids.reshape(-1).astype(jnp.int32),
        sp_table, mv_table, ab_table, it_table, te_table)
    sp2 = sp_g.reshape(B, S * EMBED)
    mv2 = mv_g.reshape(B, S * M * FEAT)
    ab2 = ab_g.reshape(B, S * FEAT)
    it2 = it_g.reshape(B, S * FEAT)
    te2 = te_g.reshape(B, S * FEAT)

    def bd(blk):
        z = jnp.zeros((S * blk.shape[0], S * POKE), f32)
        for s in range(S):
            z = z.at[s * blk.shape[0]:(s + 1) * blk.shape[0],
                     s * POKE:(s + 1) * POKE].set(blk)
        return z

    wsp = bd(W_pe[0:32])
    wmv = bd(jnp.tile(W_pe[32:48], (M, 1)))
    wab = bd(W_pe[48:64])
    wit = bd(W_pe[64:80])
    wte = bd(W_pe[80:96])
    bpe = jnp.tile(b_pe, S).reshape(1, S * POKE)

    inv = 1.0 / jnp.sqrt(1.0 + 1e-5)
    w1 = W1 * (g1 * inv)[None, :]
    b1f = (b1 * g1 * inv + be1).reshape(1, HID)
    w2 = W2 * (g2 * inv)[None, :]
    b2f = (b2 * g2 * inv + be2).reshape(1, HID)
    w3 = W3 * (g3 * inv)[None, :]
    b3f = (b3 * g3 * inv + be3).reshape(1, HID // 2)
    w1f, w1n = w1[:S * POKE], w1[S * POKE:]

    grid = (B // R,)
    row = lambda c: pl.BlockSpec((R, c), lambda i: (i, 0))
    full = lambda a: pl.BlockSpec(a.shape, lambda i: (0,) * a.ndim)
    wargs = (wsp, wmv, wab, wit, wte, bpe,
             w1f, w1n, b1f, w2, b2f, w3, b3f,
             Wv1, bv1.reshape(1, 64), Wv2, bv2.reshape(1, 1),
             Wp1[:HID // 2], Wp1[HID // 2:], bp1.reshape(1, HID // 2),
             Wp2, bp2.reshape(1, NUM_ACTIONS))
    v, p0, p1 = pl.pallas_call(
        _mlp_body,
        grid=grid,
        in_specs=[row(S * EMBED), row(S * M * FEAT), row(S * FEAT),
                  row(S * FEAT), row(S * FEAT), row(NUMERIC)]
                 + [full(a) for a in wargs],
        out_specs=[row(1), row(NUM_ACTIONS), row(NUM_ACTIONS)],
        out_shape=[jax.ShapeDtypeStruct((B, 1), f32),
                   jax.ShapeDtypeStruct((B, NUM_ACTIONS), f32),
                   jax.ShapeDtypeStruct((B, NUM_ACTIONS), f32)],
        compiler_params=pltpu.CompilerParams(
            dimension_semantics=("parallel",)),
    )(sp2, mv2, ab2, it2, te2, numeric, *wargs)
    return (v.reshape(B), p0, p1)


# speedup vs baseline: 7.1560x; 1.4511x over previous
import functools

import jax
import jax.numpy as jnp
from jax import lax
from jax.experimental import pallas as pl
from jax.experimental.pallas import tpu as pltpu
from jax.experimental.pallas import tpu_sc as plsc

B = 16384
S = 4
M = 4
EMBED = 32
FEAT = 16
POKE = 48
HID = 256
NUMERIC = 24
NUM_ACTIONS = 100

NP = B * S
NC, NS = 2, 16
NW = NC * NS
CHUNK = 256
PER_W = NP // NW
N_CHUNKS = PER_W // CHUNK
NBUF = 2

R = 512


def _sc_gather(sp_ids, mv_ids, ab_ids, it_ids, te_ids,
               sp_table, mv_table, ab_table, it_table, te_table):
    mesh = plsc.VectorSubcoreMesh(core_axis_name="c", subcore_axis_name="s")

    @functools.partial(
        pl.kernel,
        mesh=mesh,
        compiler_params=pltpu.CompilerParams(use_tc_tiling_on_sc=False),
        out_type=(
            jax.ShapeDtypeStruct((NP, EMBED), jnp.float32),
            jax.ShapeDtypeStruct((NP * M, FEAT), jnp.float32),
            jax.ShapeDtypeStruct((NP, FEAT), jnp.float32),
            jax.ShapeDtypeStruct((NP, FEAT), jnp.float32),
            jax.ShapeDtypeStruct((NP, FEAT), jnp.float32),
        ),
        scratch_types=[
            pltpu.VMEM((PER_W,), jnp.int32),
            pltpu.VMEM((PER_W * M,), jnp.int32),
            pltpu.VMEM((PER_W,), jnp.int32),
            pltpu.VMEM((PER_W,), jnp.int32),
            pltpu.VMEM((PER_W,), jnp.int32),
            pltpu.VMEM((NBUF, CHUNK, EMBED), jnp.float32),
            pltpu.VMEM((NBUF, CHUNK * M, FEAT), jnp.float32),
            pltpu.VMEM((NBUF, CHUNK, FEAT), jnp.float32),
            pltpu.VMEM((NBUF, CHUNK, FEAT), jnp.float32),
            pltpu.VMEM((NBUF, CHUNK, FEAT), jnp.float32),
            pltpu.SemaphoreType.DMA((NBUF,)),
            pltpu.SemaphoreType.DMA((NBUF,)),
            pltpu.SemaphoreType.DMA,
            pltpu.VMEM_SHARED((1500, EMBED), jnp.float32),
            pltpu.VMEM_SHARED((1000, FEAT), jnp.float32),
            pltpu.VMEM_SHARED((400, FEAT), jnp.float32),
            pltpu.VMEM_SHARED((600, FEAT), jnp.float32),
            pltpu.VMEM_SHARED((20, FEAT), jnp.float32),
        ],
    )
    def k(sp_ids_h, mv_ids_h, ab_ids_h, it_ids_h, te_ids_h,
          sp_th, mv_th, ab_th, it_th, te_th,
          sp_o, mv_o, ab_o, it_o, te_o,
          i_sp, i_mv, i_ab, i_it, i_te,
          d_sp, d_mv, d_ab, d_it, d_te, gsem, wsem, isem,
          sp_t, mv_t, ab_t, it_t, te_t):
        sid = lax.axis_index("s")
        wid = sid * NC + lax.axis_index("c")
        w0 = wid * PER_W
        @pl.when(sid == 0)
        def _():
            tcps = [
                pltpu.async_copy(sp_th, sp_t, isem),
                pltpu.async_copy(mv_th, mv_t, isem),
                pltpu.async_copy(ab_th, ab_t, isem),
                pltpu.async_copy(it_th, it_t, isem),
                pltpu.async_copy(te_th, te_t, isem),
            ]
            for cp in tcps:
                cp.wait()
        plsc.subcore_barrier()
        idcps = [
            pltpu.async_copy(sp_ids_h.at[pl.ds(w0, PER_W)], i_sp, isem),
            pltpu.async_copy(mv_ids_h.at[pl.ds(w0 * M, PER_W * M)], i_mv, isem),
            pltpu.async_copy(ab_ids_h.at[pl.ds(w0, PER_W)], i_ab, isem),
            pltpu.async_copy(it_ids_h.at[pl.ds(w0, PER_W)], i_it, isem),
            pltpu.async_copy(te_ids_h.at[pl.ds(w0, PER_W)], i_te, isem),
        ]
        for cp in idcps:
            cp.wait()

        def fire_gather(c):
            b = c % NBUF
            o = c * CHUNK
            return [
                pltpu.async_copy(sp_t.at[i_sp.at[pl.ds(o, CHUNK)]],
                                 d_sp.at[b], gsem.at[b]),
                pltpu.async_copy(mv_t.at[i_mv.at[pl.ds(o * M, CHUNK * M)]],
                                 d_mv.at[b], gsem.at[b]),
                pltpu.async_copy(ab_t.at[i_ab.at[pl.ds(o, CHUNK)]],
                                 d_ab.at[b], gsem.at[b]),
                pltpu.async_copy(it_t.at[i_it.at[pl.ds(o, CHUNK)]],
                                 d_it.at[b], gsem.at[b]),
                pltpu.async_copy(te_t.at[i_te.at[pl.ds(o, CHUNK)]],
                                 d_te.at[b], gsem.at[b]),
            ]

        def fire_write(c):
            b = c % NBUF
            base = w0 + c * CHUNK
            return [
                pltpu.async_copy(d_sp.at[b], sp_o.at[pl.ds(base, CHUNK)],
                                 wsem.at[b]),
                pltpu.async_copy(d_mv.at[b], mv_o.at[pl.ds(base * M, CHUNK * M)],
                                 wsem.at[b]),
                pltpu.async_copy(d_ab.at[b], ab_o.at[pl.ds(base, CHUNK)],
                                 wsem.at[b]),
                pltpu.async_copy(d_it.at[b], it_o.at[pl.ds(base, CHUNK)],
                                 wsem.at[b]),
                pltpu.async_copy(d_te.at[b], te_o.at[pl.ds(base, CHUNK)],
                                 wsem.at[b]),
            ]

        gcps = {c: fire_gather(c) for c in range(min(NBUF, N_CHUNKS))}
        wcps = {}
        for c in range(N_CHUNKS):
            for cp in gcps.pop(c):
                cp.wait()
            wcps[c] = fire_write(c)
            nxt = c + NBUF
            if nxt < N_CHUNKS:
                for cp in wcps.pop(c):
                    cp.wait()
                gcps[nxt] = fire_gather(nxt)
        for c, cps in wcps.items():
            for cp in cps:
                cp.wait()

    return k(sp_ids, mv_ids, ab_ids, it_ids, te_ids,
             sp_table, mv_table, ab_table, it_table, te_table)


def _mlp_body(sp_r, mv_r, ab_r, it_r, te_r, num_r,
              wsp_r, wmv_r, wab_r, wit_r, wte_r, bpe_r,
              w1f_r, w1n_r, b1_r, w2_r, b2_r, w3_r, b3_r,
              wv1_r, bv1_r, wv2_r, bv2_r,
              wp1x_r, wp1s_r, bp1_r, wp2_r, bp2_r,
              v_o, p0_o, p1_o):
    dot = functools.partial(jnp.dot, preferred_element_type=jnp.float32)
    enc = dot(sp_r[:], wsp_r[:]) + dot(mv_r[:], wmv_r[:]) \
        + dot(ab_r[:], wab_r[:]) + dot(it_r[:], wit_r[:]) \
        + dot(te_r[:], wte_r[:]) + bpe_r[:]
    enc = jnp.maximum(enc, 0.0)
    x = jnp.maximum(dot(enc, w1f_r[:]) + dot(num_r[:], w1n_r[:]) + b1_r[:], 0.0)
    x = jnp.maximum(dot(x, w2_r[:]) + b2_r[:], 0.0)
    x = jnp.maximum(dot(x, w3_r[:]) + b3_r[:], 0.0)
    v = dot(jnp.maximum(dot(x, wv1_r[:]) + bv1_r[:], 0.0), wv2_r[:]) + bv2_r[:]
    v_o[:] = v
    for i, p_o in ((0, p0_o), (1, p1_o)):
        slot = enc[:, i * POKE:(i + 1) * POKE]
        h = jnp.maximum(dot(x, wp1x_r[:]) + dot(slot, wp1s_r[:]) + bp1_r[:], 0.0)
        p_o[:] = dot(h, wp2_r[:]) + bp2_r[:]


def kernel(species_ids, move_ids, ability_ids, item_ids, tera_ids, numeric, sp_table, mv_table, ab_table, it_table, te_table, W_pe, b_pe, W1, b1, g1, be1, W2, b2, g2, be2, W3, b3, g3, be3, Wv1, bv1, Wv2, bv2, Wp1, bp1, Wp2, bp2):
    f32 = jnp.float32
    sp_g, mv_g, ab_g, it_g, te_g = _sc_gather(
        species_ids.reshape(-1).astype(jnp.int32),
        move_ids.reshape(-1).astype(jnp.int32),
        ability_ids.reshape(-1).astype(jnp.int32),
        item_ids.reshape(-1).astype(jnp.int32),
        tera_ids.reshape(-1).astype(jnp.int32),
        sp_table, mv_table, ab_table, it_table, te_table)
    sp2 = sp_g.reshape(B, S * EMBED)
    mv2 = mv_g.reshape(B, S * M * FEAT)
    ab2 = ab_g.reshape(B, S * FEAT)
    it2 = it_g.reshape(B, S * FEAT)
    te2 = te_g.reshape(B, S * FEAT)

    def bd(blk):
        z = jnp.zeros((S * blk.shape[0], S * POKE), f32)
        for s in range(S):
            z = z.at[s * blk.shape[0]:(s + 1) * blk.shape[0],
                     s * POKE:(s + 1) * POKE].set(blk)
        return z

    wsp = bd(W_pe[0:32])
    wmv = bd(jnp.tile(W_pe[32:48], (M, 1)))
    wab = bd(W_pe[48:64])
    wit = bd(W_pe[64:80])
    wte = bd(W_pe[80:96])
    bpe = jnp.tile(b_pe, S).reshape(1, S * POKE)

    inv = 1.0 / jnp.sqrt(1.0 + 1e-5)
    w1 = W1 * (g1 * inv)[None, :]
    b1f = (b1 * g1 * inv + be1).reshape(1, HID)
    w2 = W2 * (g2 * inv)[None, :]
    b2f = (b2 * g2 * inv + be2).reshape(1, HID)
    w3 = W3 * (g3 * inv)[None, :]
    b3f = (b3 * g3 * inv + be3).reshape(1, HID // 2)
    w1f, w1n = w1[:S * POKE], w1[S * POKE:]

    grid = (B // R,)
    row = lambda c: pl.BlockSpec((R, c), lambda i: (i, 0))
    full = lambda a: pl.BlockSpec(a.shape, lambda i: (0,) * a.ndim)
    wargs = (wsp, wmv, wab, wit, wte, bpe,
             w1f, w1n, b1f, w2, b2f, w3, b3f,
             Wv1, bv1.reshape(1, 64), Wv2, bv2.reshape(1, 1),
             Wp1[:HID // 2], Wp1[HID // 2:], bp1.reshape(1, HID // 2),
             Wp2, bp2.reshape(1, NUM_ACTIONS))
    v, p0, p1 = pl.pallas_call(
        _mlp_body,
        grid=grid,
        in_specs=[row(S * EMBED), row(S * M * FEAT), row(S * FEAT),
                  row(S * FEAT), row(S * FEAT), row(NUMERIC)]
                 + [full(a) for a in wargs],
        out_specs=[row(1), row(NUM_ACTIONS), row(NUM_ACTIONS)],
        out_shape=[jax.ShapeDtypeStruct((B, 1), f32),
                   jax.ShapeDtypeStruct((B, NUM_ACTIONS), f32),
                   jax.ShapeDtypeStruct((B, NUM_ACTIONS), f32)],
        compiler_params=pltpu.CompilerParams(
            dimension_semantics=("parallel",)),
    )(sp2, mv2, ab2, it2, te2, numeric, *wargs)
    return (v.reshape(B), p0, p1)


# speedup vs baseline: 7.8480x; 1.0967x over previous
import functools

import jax
import jax.numpy as jnp
from jax import lax
from jax.experimental import pallas as pl
from jax.experimental.pallas import tpu as pltpu
from jax.experimental.pallas import tpu_sc as plsc

B = 16384
S = 4
M = 4
EMBED = 32
FEAT = 16
POKE = 48
HID = 256
NUMERIC = 24
NUM_ACTIONS = 100

NC, NS = 2, 16
NW = NC * NS
BPW = B // NW
CB = 32
NCH = BPW // CB
NBUF = 2

R = 512


def _sc_gather(sp1d, mv1d, ab1d, it1d, te1d,
               sp_table, mv_table, ab_table, it_table, te_table):
    mesh = plsc.VectorSubcoreMesh(core_axis_name="c", subcore_axis_name="s")

    @functools.partial(
        pl.kernel,
        mesh=mesh,
        compiler_params=pltpu.CompilerParams(use_tc_tiling_on_sc=False),
        out_type=(
            jax.ShapeDtypeStruct((B, 128), jnp.float32),
            jax.ShapeDtypeStruct((B, 128), jnp.float32),
            jax.ShapeDtypeStruct((B, 128), jnp.float32),
        ),
        scratch_types=[
            pltpu.VMEM((BPW * S,), jnp.int32),
            pltpu.VMEM((BPW * S * M,), jnp.int32),
            pltpu.VMEM((BPW * S,), jnp.int32),
            pltpu.VMEM((BPW * S,), jnp.int32),
            pltpu.VMEM((BPW * S,), jnp.int32),
            pltpu.VMEM((NBUF, CB * S, EMBED), jnp.float32),
            pltpu.VMEM((NBUF, CB * S * M, FEAT), jnp.float32),
            pltpu.VMEM((NBUF, CB * S, FEAT), jnp.float32),
            pltpu.VMEM((NBUF, CB * S, FEAT), jnp.float32),
            pltpu.VMEM((NBUF, CB * S, FEAT), jnp.float32),
            pltpu.VMEM((NBUF, CB, 128), jnp.float32),
            pltpu.VMEM((NBUF, CB, 128), jnp.float32),
            pltpu.VMEM((NBUF, CB, 128), jnp.float32),
            pltpu.SemaphoreType.DMA((NBUF,)),
            pltpu.SemaphoreType.DMA((NBUF,)),
            pltpu.SemaphoreType.DMA,
            pltpu.VMEM_SHARED((1500, EMBED), jnp.float32),
            pltpu.VMEM_SHARED((1000, FEAT), jnp.float32),
            pltpu.VMEM_SHARED((400, FEAT), jnp.float32),
            pltpu.VMEM_SHARED((600, FEAT), jnp.float32),
            pltpu.VMEM_SHARED((20, FEAT), jnp.float32),
        ],
    )
    def k(sp_ids_h, mv_ids_h, ab_ids_h, it_ids_h, te_ids_h,
          sp_th, mv_th, ab_th, it_th, te_th,
          sp_o, mvte_o, abit_o,
          i_sp, i_mv, i_ab, i_it, i_te,
          d_sp, d_mv, d_ab, d_it, d_te, o_sp, o_mvte, o_abit,
          gsem, wsem, isem,
          sp_t, mv_t, ab_t, it_t, te_t):
        sid = lax.axis_index("s")
        wid = sid * NC + lax.axis_index("c")
        w0 = wid * BPW

        @pl.when(sid == 0)
        def _():
            tcps = [
                pltpu.async_copy(sp_th, sp_t, isem),
                pltpu.async_copy(mv_th, mv_t, isem),
                pltpu.async_copy(ab_th, ab_t, isem),
                pltpu.async_copy(it_th, it_t, isem),
                pltpu.async_copy(te_th, te_t, isem),
            ]
            for cp in tcps:
                cp.wait()

        idcps = [
            pltpu.async_copy(sp_ids_h.at[pl.ds(w0 * S, BPW * S)], i_sp, isem),
            pltpu.async_copy(mv_ids_h.at[pl.ds(w0 * S * M, BPW * S * M)], i_mv, isem),
            pltpu.async_copy(ab_ids_h.at[pl.ds(w0 * S, BPW * S)], i_ab, isem),
            pltpu.async_copy(it_ids_h.at[pl.ds(w0 * S, BPW * S)], i_it, isem),
            pltpu.async_copy(te_ids_h.at[pl.ds(w0 * S, BPW * S)], i_te, isem),
        ]
        for cp in idcps:
            cp.wait()
        plsc.subcore_barrier()

        def fire_gather(c):
            b = c % NBUF
            o, om = c * CB * S, c * CB * S * M
            return [
                pltpu.async_copy(sp_t.at[i_sp.at[pl.ds(o, CB * S)]],
                                 d_sp.at[b], gsem.at[b]),
                pltpu.async_copy(mv_t.at[i_mv.at[pl.ds(om, CB * S * M)]],
                                 d_mv.at[b], gsem.at[b]),
                pltpu.async_copy(ab_t.at[i_ab.at[pl.ds(o, CB * S)]],
                                 d_ab.at[b], gsem.at[b]),
                pltpu.async_copy(it_t.at[i_it.at[pl.ds(o, CB * S)]],
                                 d_it.at[b], gsem.at[b]),
                pltpu.async_copy(te_t.at[i_te.at[pl.ds(o, CB * S)]],
                                 d_te.at[b], gsem.at[b]),
            ]

        def repack(c):
            b = c % NBUF
            dsp, dmv = d_sp.at[b], d_mv.at[b]
            dab, dit, dte = d_ab.at[b], d_it.at[b], d_te.at[b]
            osp, omv, oab = o_sp.at[b], o_mvte.at[b], o_abit.at[b]

            def body(r, carry):
                for s in range(S):
                    p = r * S + s
                    acc = (dmv[p * M] + dmv[p * M + 1]
                           + dmv[p * M + 2] + dmv[p * M + 3])
                    omv[r, pl.ds(16 * s, 16)] = acc
                    omv[r, pl.ds(64 + 16 * s, 16)] = dte[p]
                    oab[r, pl.ds(16 * s, 16)] = dab[p]
                    oab[r, pl.ds(64 + 16 * s, 16)] = dit[p]
                    osp[r, pl.ds(32 * s, 16)] = dsp[p, pl.ds(0, 16)]
                    osp[r, pl.ds(32 * s + 16, 16)] = dsp[p, pl.ds(16, 16)]
                return carry

            lax.fori_loop(0, CB, body, 0)

        def fire_write(c):
            b = c % NBUF
            r0 = w0 + c * CB
            return [
                pltpu.async_copy(o_sp.at[b], sp_o.at[pl.ds(r0, CB)], wsem.at[b]),
                pltpu.async_copy(o_mvte.at[b], mvte_o.at[pl.ds(r0, CB)], wsem.at[b]),
                pltpu.async_copy(o_abit.at[b], abit_o.at[pl.ds(r0, CB)], wsem.at[b]),
            ]

        gcps = {c: fire_gather(c) for c in range(min(NBUF, NCH))}
        wcps = {}
        for c in range(NCH):
            for cp in gcps.pop(c):
                cp.wait()
            repack(c)
            wcps[c] = fire_write(c)
            nxt = c + NBUF
            if nxt < NCH:
                for cp in wcps.pop(c):
                    cp.wait()
                gcps[nxt] = fire_gather(nxt)
        for c, cps in wcps.items():
            for cp in cps:
                cp.wait()

    return k(sp1d, mv1d, ab1d, it1d, te1d,
             sp_table, mv_table, ab_table, it_table, te_table)


def _mlp_body(sp_r, mvte_r, abit_r, num_r,
              wsp_r, wmvte_r, wabit_r, bpe_r,
              w1f_r, w1n_r, b1_r, w2_r, b2_r, w3_r, b3_r,
              wv1_r, bv1_r, wv2_r, bv2_r,
              wp1x_r, wp1s_r, bp1_r, wp2_r, bp2_r,
              v_o, p0_o, p1_o):
    dot = functools.partial(jnp.dot, preferred_element_type=jnp.float32)
    enc = dot(sp_r[:], wsp_r[:]) + dot(mvte_r[:], wmvte_r[:]) \
        + dot(abit_r[:], wabit_r[:]) + bpe_r[:]
    enc = jnp.maximum(enc, 0.0)
    x = jnp.maximum(dot(enc, w1f_r[:]) + dot(num_r[:], w1n_r[:]) + b1_r[:], 0.0)
    x = jnp.maximum(dot(x, w2_r[:]) + b2_r[:], 0.0)
    x = jnp.maximum(dot(x, w3_r[:]) + b3_r[:], 0.0)
    v = dot(jnp.maximum(dot(x, wv1_r[:]) + bv1_r[:], 0.0), wv2_r[:]) + bv2_r[:]
    v_o[:] = v
    for i, p_o in ((0, p0_o), (1, p1_o)):
        slot = enc[:, i * POKE:(i + 1) * POKE]
        h = jnp.maximum(dot(x, wp1x_r[:]) + dot(slot, wp1s_r[:]) + bp1_r[:], 0.0)
        p_o[:] = dot(h, wp2_r[:]) + bp2_r[:]


def kernel(species_ids, move_ids, ability_ids, item_ids, tera_ids, numeric, sp_table, mv_table, ab_table, it_table, te_table, W_pe, b_pe, W1, b1, g1, be1, W2, b2, g2, be2, W3, b3, g3, be3, Wv1, bv1, Wv2, bv2, Wp1, bp1, Wp2, bp2):
    f32 = jnp.float32
    sp2, mvte, abit = _sc_gather(
        species_ids.reshape(-1), move_ids.reshape(-1), ability_ids.reshape(-1),
        item_ids.reshape(-1), tera_ids.reshape(-1),
        sp_table, mv_table, ab_table, it_table, te_table)

    def bd16(blk_lo, blk_hi):
        z = jnp.zeros((128, S * POKE), f32)
        for s in range(S):
            z = z.at[16 * s:16 * (s + 1), POKE * s:POKE * (s + 1)].set(blk_lo)
            z = z.at[64 + 16 * s:64 + 16 * (s + 1),
                     POKE * s:POKE * (s + 1)].set(blk_hi)
        return z

    wsp = jnp.zeros((128, S * POKE), f32)
    for s in range(S):
        wsp = wsp.at[32 * s:32 * (s + 1), POKE * s:POKE * (s + 1)].set(W_pe[0:32])
    wmvte = bd16(W_pe[32:48], W_pe[80:96])
    wabit = bd16(W_pe[48:64], W_pe[64:80])
    bpe = jnp.tile(b_pe, S).reshape(1, S * POKE)

    inv = 1.0 / jnp.sqrt(1.0 + 1e-5)
    w1 = W1 * (g1 * inv)[None, :]
    b1f = (b1 * g1 * inv + be1).reshape(1, HID)
    w2 = W2 * (g2 * inv)[None, :]
    b2f = (b2 * g2 * inv + be2).reshape(1, HID)
    w3 = W3 * (g3 * inv)[None, :]
    b3f = (b3 * g3 * inv + be3).reshape(1, HID // 2)
    w1f, w1n = w1[:S * POKE], w1[S * POKE:]

    grid = (B // R,)
    row = lambda c: pl.BlockSpec((R, c), lambda i: (i, 0))
    full = lambda a: pl.BlockSpec(a.shape, lambda i: (0,) * a.ndim)
    wargs = (wsp, wmvte, wabit, bpe,
             w1f, w1n, b1f, w2, b2f, w3, b3f,
             Wv1, bv1.reshape(1, 64), Wv2, bv2.reshape(1, 1),
             Wp1[:HID // 2], Wp1[HID // 2:], bp1.reshape(1, HID // 2),
             Wp2, bp2.reshape(1, NUM_ACTIONS))
    v, p0, p1 = pl.pallas_call(
        _mlp_body,
        grid=grid,
        in_specs=[row(128), row(128), row(128), row(NUMERIC)]
                 + [full(a) for a in wargs],
        out_specs=[row(1), row(NUM_ACTIONS), row(NUM_ACTIONS)],
        out_shape=[jax.ShapeDtypeStruct((B, 1), f32),
                   jax.ShapeDtypeStruct((B, NUM_ACTIONS), f32),
                   jax.ShapeDtypeStruct((B, NUM_ACTIONS), f32)],
        compiler_params=pltpu.CompilerParams(
            dimension_semantics=("parallel",)),
    )(sp2, mvte, abit, numeric, *wargs)
    return (v.reshape(B), p0, p1)


# speedup vs baseline: 8.3815x; 1.0680x over previous
import functools

import jax
import jax.numpy as jnp
from jax import lax
from jax.experimental import pallas as pl
from jax.experimental.pallas import tpu as pltpu
from jax.experimental.pallas import tpu_sc as plsc

B = 16384
S = 4
M = 4
EMBED = 32
FEAT = 16
POKE = 48
HID = 256
NUMERIC = 24
NUM_ACTIONS = 100

NC, NS = 2, 16
NW = NC * NS
BPW = B // NW
CB = 64
NCH = BPW // CB
NBUF = 2

R = 512

_SC_MESH = dict(core_axis_name="c", subcore_axis_name="s")


def _sc_gather_a(sp1d, ab1d, it1d, te1d, sp_table, ab_table, it_table, te_table):
    @functools.partial(
        pl.kernel,
        mesh=plsc.VectorSubcoreMesh(**_SC_MESH),
        compiler_params=pltpu.CompilerParams(use_tc_tiling_on_sc=False),
        out_type=(
            jax.ShapeDtypeStruct((B, 128), jnp.float32),
            jax.ShapeDtypeStruct((B, 128), jnp.float32),
            jax.ShapeDtypeStruct((B, 128), jnp.float32),
        ),
        scratch_types=[
            pltpu.VMEM((BPW * S,), jnp.int32),
            pltpu.VMEM((BPW * S,), jnp.int32),
            pltpu.VMEM((BPW * S,), jnp.int32),
            pltpu.VMEM((BPW * S,), jnp.int32),
            pltpu.VMEM((NBUF, CB * S, EMBED), jnp.float32),
            pltpu.VMEM((NBUF, CB * S, FEAT), jnp.float32),
            pltpu.VMEM((NBUF, CB * S, FEAT), jnp.float32),
            pltpu.VMEM((NBUF, CB * S, FEAT), jnp.float32),
            pltpu.VMEM((NBUF, CB, 128), jnp.float32),
            pltpu.VMEM((NBUF, CB, 128), jnp.float32),
            pltpu.VMEM((NBUF, CB, 128), jnp.float32),
            pltpu.SemaphoreType.DMA((NBUF,)),
            pltpu.SemaphoreType.DMA((NBUF,)),
            pltpu.SemaphoreType.DMA,
            pltpu.VMEM_SHARED((1500, EMBED), jnp.float32),
            pltpu.VMEM_SHARED((400, FEAT), jnp.float32),
            pltpu.VMEM_SHARED((600, FEAT), jnp.float32),
            pltpu.VMEM_SHARED((20, FEAT), jnp.float32),
        ],
    )
    def k(sp_ids_h, ab_ids_h, it_ids_h, te_ids_h,
          sp_th, ab_th, it_th, te_th,
          sp_o, abit_o, tep_o,
          i_sp, i_ab, i_it, i_te,
          d_sp, d_ab, d_it, d_te, o_sp, o_abit, o_tep,
          gsem, wsem, isem,
          sp_t, ab_t, it_t, te_t):
        sid = lax.axis_index("s")
        wid = sid * NC + lax.axis_index("c")
        w0 = wid * BPW

        @pl.when(sid == 0)
        def _():
            tcps = [
                pltpu.async_copy(sp_th, sp_t, isem),
                pltpu.async_copy(ab_th, ab_t, isem),
                pltpu.async_copy(it_th, it_t, isem),
                pltpu.async_copy(te_th, te_t, isem),
            ]
            for cp in tcps:
                cp.wait()

        idcps = [
            pltpu.async_copy(sp_ids_h.at[pl.ds(w0 * S, BPW * S)], i_sp, isem),
            pltpu.async_copy(ab_ids_h.at[pl.ds(w0 * S, BPW * S)], i_ab, isem),
            pltpu.async_copy(it_ids_h.at[pl.ds(w0 * S, BPW * S)], i_it, isem),
            pltpu.async_copy(te_ids_h.at[pl.ds(w0 * S, BPW * S)], i_te, isem),
        ]
        for cp in idcps:
            cp.wait()
        plsc.subcore_barrier()

        def fire_gather(c):
            b = c % NBUF
            o = c * CB * S
            return [
                pltpu.async_copy(sp_t.at[i_sp.at[pl.ds(o, CB * S)]],
                                 d_sp.at[b], gsem.at[b]),
                pltpu.async_copy(ab_t.at[i_ab.at[pl.ds(o, CB * S)]],
                                 d_ab.at[b], gsem.at[b]),
                pltpu.async_copy(it_t.at[i_it.at[pl.ds(o, CB * S)]],
                                 d_it.at[b], gsem.at[b]),
                pltpu.async_copy(te_t.at[i_te.at[pl.ds(o, CB * S)]],
                                 d_te.at[b], gsem.at[b]),
            ]

        def repack(c):
            b = c % NBUF
            dsp, dab = d_sp.at[b], d_ab.at[b]
            dit, dte = d_it.at[b], d_te.at[b]
            osp, oab, ote = o_sp.at[b], o_abit.at[b], o_tep.at[b]
            zero = jnp.zeros((16,), jnp.float32)

            def body(r, carry):
                for s in range(S):
                    p = r * S + s
                    osp[r, pl.ds(32 * s, 16)] = dsp[p, pl.ds(0, 16)]
                    osp[r, pl.ds(32 * s + 16, 16)] = dsp[p, pl.ds(16, 16)]
                    oab[r, pl.ds(16 * s, 16)] = dab[p]
                    oab[r, pl.ds(64 + 16 * s, 16)] = dit[p]
                    ote[r, pl.ds(16 * s, 16)] = dte[p]
                    ote[r, pl.ds(64 + 16 * s, 16)] = zero
                return carry

            lax.fori_loop(0, CB, body, 0)

        def fire_write(c):
            b = c % NBUF
            r0 = w0 + c * CB
            return [
                pltpu.async_copy(o_sp.at[b], sp_o.at[pl.ds(r0, CB)], wsem.at[b]),
                pltpu.async_copy(o_abit.at[b], abit_o.at[pl.ds(r0, CB)], wsem.at[b]),
                pltpu.async_copy(o_tep.at[b], tep_o.at[pl.ds(r0, CB)], wsem.at[b]),
            ]

        gcps = {c: fire_gather(c) for c in range(min(NBUF, NCH))}
        wcps = {}
        for c in range(NCH):
            for cp in gcps.pop(c):
                cp.wait()
            repack(c)
            wcps[c] = fire_write(c)
            nxt = c + NBUF
            if nxt < NCH:
                for cp in wcps.pop(c):
                    cp.wait()
                gcps[nxt] = fire_gather(nxt)
        for c, cps in wcps.items():
            for cp in cps:
                cp.wait()

    return k(sp1d, ab1d, it1d, te1d, sp_table, ab_table, it_table, te_table)


def _sc_gather_b(mv1d, mv_table):
    @functools.partial(
        pl.kernel,
        mesh=plsc.VectorSubcoreMesh(**_SC_MESH),
        compiler_params=pltpu.CompilerParams(use_tc_tiling_on_sc=False),
        out_type=jax.ShapeDtypeStruct((B, 128), jnp.float32),
        scratch_types=[
            pltpu.VMEM((BPW * S * M,), jnp.int32),
            pltpu.VMEM((NBUF, CB * S * M, FEAT), jnp.float32),
            pltpu.VMEM((NBUF, CB, 128), jnp.float32),
            pltpu.SemaphoreType.DMA((NBUF,)),
            pltpu.SemaphoreType.DMA((NBUF,)),
            pltpu.SemaphoreType.DMA,
            pltpu.VMEM_SHARED((1000, FEAT), jnp.float32),
        ],
    )
    def k(mv_ids_h, mv_th, mvp_o, i_mv, d_mv, o_mv, gsem, wsem, isem, mv_t):
        sid = lax.axis_index("s")
        wid = sid * NC + lax.axis_index("c")
        w0 = wid * BPW

        @pl.when(sid == 0)
        def _():
            pltpu.async_copy(mv_th, mv_t, isem).wait()

        pltpu.async_copy(
            mv_ids_h.at[pl.ds(w0 * S * M, BPW * S * M)], i_mv, isem).wait()
        plsc.subcore_barrier()

        def fire_gather(c):
            b = c % NBUF
            om = c * CB * S * M
            return [pltpu.async_copy(mv_t.at[i_mv.at[pl.ds(om, CB * S * M)]],
                                     d_mv.at[b], gsem.at[b])]

        def repack(c):
            b = c % NBUF
            dmv, omv = d_mv.at[b], o_mv.at[b]
            zero = jnp.zeros((16,), jnp.float32)

            def body(r, carry):
                for s in range(S):
                    p = (r * S + s) * M
                    acc = dmv[p] + dmv[p + 1] + dmv[p + 2] + dmv[p + 3]
                    omv[r, pl.ds(16 * s, 16)] = acc
                    omv[r, pl.ds(64 + 16 * s, 16)] = zero
                return carry

            lax.fori_loop(0, CB, body, 0)

        def fire_write(c):
            b = c % NBUF
            return [pltpu.async_copy(o_mv.at[b],
                                     mvp_o.at[pl.ds(w0 + c * CB, CB)],
                                     wsem.at[b])]

        gcps = {c: fire_gather(c) for c in range(min(NBUF, NCH))}
        wcps = {}
        for c in range(NCH):
            for cp in gcps.pop(c):
                cp.wait()
            repack(c)
            wcps[c] = fire_write(c)
            nxt = c + NBUF
            if nxt < NCH:
                for cp in wcps.pop(c):
                    cp.wait()
                gcps[nxt] = fire_gather(nxt)
        for c, cps in wcps.items():
            for cp in cps:
                cp.wait()

    return k(mv1d, mv_table)


def _mlp_body(sp_r, abit_r, tep_r, mvp_r, num_r,
              wsp_r, wabit_r, wtep_r, wmvp_r, bpe_r,
              w1f_r, w1n_r, b1_r, w2_r, b2_r, w3_r, b3_r,
              wv1_r, bv1_r, wv2_r, bv2_r,
              wp1x_r, wp1s_r, bp1_r, wp2_r, bp2_r,
              v_o, p0_o, p1_o):
    dot = functools.partial(jnp.dot, preferred_element_type=jnp.float32)
    enc = dot(sp_r[:], wsp_r[:]) + dot(abit_r[:], wabit_r[:]) \
        + dot(tep_r[:], wtep_r[:]) + dot(mvp_r[:], wmvp_r[:]) + bpe_r[:]
    enc = jnp.maximum(enc, 0.0)
    x = jnp.maximum(dot(enc, w1f_r[:]) + dot(num_r[:], w1n_r[:]) + b1_r[:], 0.0)
    x = jnp.maximum(dot(x, w2_r[:]) + b2_r[:], 0.0)
    x = jnp.maximum(dot(x, w3_r[:]) + b3_r[:], 0.0)
    v = dot(jnp.maximum(dot(x, wv1_r[:]) + bv1_r[:], 0.0), wv2_r[:]) + bv2_r[:]
    v_o[:] = v
    for i, p_o in ((0, p0_o), (1, p1_o)):
        slot = enc[:, i * POKE:(i + 1) * POKE]
        h = jnp.maximum(dot(x, wp1x_r[:]) + dot(slot, wp1s_r[:]) + bp1_r[:], 0.0)
        p_o[:] = dot(h, wp2_r[:]) + bp2_r[:]


def kernel(species_ids, move_ids, ability_ids, item_ids, tera_ids, numeric, sp_table, mv_table, ab_table, it_table, te_table, W_pe, b_pe, W1, b1, g1, be1, W2, b2, g2, be2, W3, b3, g3, be3, Wv1, bv1, Wv2, bv2, Wp1, bp1, Wp2, bp2):
    f32 = jnp.float32
    sp2, abit, tep = _sc_gather_a(
        species_ids.reshape(-1), ability_ids.reshape(-1),
        item_ids.reshape(-1), tera_ids.reshape(-1),
        sp_table, ab_table, it_table, te_table)
    mv1d, sp2 = lax.optimization_barrier((move_ids.reshape(-1), sp2))
    mvp = _sc_gather_b(mv1d, mv_table)

    def bd16(blk_lo, blk_hi):
        z = jnp.zeros((128, S * POKE), f32)
        for s in range(S):
            z = z.at[16 * s:16 * (s + 1), POKE * s:POKE * (s + 1)].set(blk_lo)
            if blk_hi is not None:
                z = z.at[64 + 16 * s:64 + 16 * (s + 1),
                         POKE * s:POKE * (s + 1)].set(blk_hi)
        return z

    wsp = jnp.zeros((128, S * POKE), f32)
    for s in range(S):
        wsp = wsp.at[32 * s:32 * (s + 1), POKE * s:POKE * (s + 1)].set(W_pe[0:32])
    wabit = bd16(W_pe[48:64], W_pe[64:80])
    wtep = bd16(W_pe[80:96], None)
    wmvp = bd16(W_pe[32:48], None)
    bpe = jnp.tile(b_pe, S).reshape(1, S * POKE)

    inv = 1.0 / jnp.sqrt(1.0 + 1e-5)
    w1 = W1 * (g1 * inv)[None, :]
    b1f = (b1 * g1 * inv + be1).reshape(1, HID)
    w2 = W2 * (g2 * inv)[None, :]
    b2f = (b2 * g2 * inv + be2).reshape(1, HID)
    w3 = W3 * (g3 * inv)[None, :]
    b3f = (b3 * g3 * inv + be3).reshape(1, HID // 2)
    w1f, w1n = w1[:S * POKE], w1[S * POKE:]

    grid = (B // R,)
    row = lambda c: pl.BlockSpec((R, c), lambda i: (i, 0))
    full = lambda a: pl.BlockSpec(a.shape, lambda i: (0,) * a.ndim)
    wargs = (wsp, wabit, wtep, wmvp, bpe,
             w1f, w1n, b1f, w2, b2f, w3, b3f,
             Wv1, bv1.reshape(1, 64), Wv2, bv2.reshape(1, 1),
             Wp1[:HID // 2], Wp1[HID // 2:], bp1.reshape(1, HID // 2),
             Wp2, bp2.reshape(1, NUM_ACTIONS))
    v, p0, p1 = pl.pallas_call(
        _mlp_body,
        grid=grid,
        in_specs=[row(128), row(128), row(128), row(128), row(NUMERIC)]
                 + [full(a) for a in wargs],
        out_specs=[row(1), row(NUM_ACTIONS), row(NUM_ACTIONS)],
        out_shape=[jax.ShapeDtypeStruct((B, 1), f32),
                   jax.ShapeDtypeStruct((B, NUM_ACTIONS), f32),
                   jax.ShapeDtypeStruct((B, NUM_ACTIONS), f32)],
        compiler_params=pltpu.CompilerParams(
            dimension_semantics=("parallel",)),
    )(sp2, abit, tep, mvp, numeric, *wargs)
    return (v.reshape(B), p0, p1)


# speedup vs baseline: 8.5934x; 1.0253x over previous
import functools

import jax
import jax.numpy as jnp
from jax import lax
from jax.experimental import pallas as pl
from jax.experimental.pallas import tpu as pltpu
from jax.experimental.pallas import tpu_sc as plsc

B = 16384
S = 4
M = 4
EMBED = 32
FEAT = 16
POKE = 48
HID = 256
NUMERIC = 24
NUM_ACTIONS = 100

NC, NS = 2, 16
NW = NC * NS
BPW = B // NW
CB = 64
NCH = BPW // CB
NBUF = 2

R = 512

_SC_MESH = dict(core_axis_name="c", subcore_axis_name="s")


def _sc_gather_a(sp1d, ab1d, it1d, te1d, sp_table, ab_table, it_table, te_table):
    @functools.partial(
        pl.kernel,
        mesh=plsc.VectorSubcoreMesh(**_SC_MESH),
        compiler_params=pltpu.CompilerParams(use_tc_tiling_on_sc=False),
        out_type=(
            jax.ShapeDtypeStruct((B, 128), jnp.float32),
            jax.ShapeDtypeStruct((B, 128), jnp.float32),
            jax.ShapeDtypeStruct((B, 128), jnp.float32),
        ),
        scratch_types=[
            pltpu.VMEM((BPW * S,), jnp.int32),
            pltpu.VMEM((BPW * S,), jnp.int32),
            pltpu.VMEM((BPW * S,), jnp.int32),
            pltpu.VMEM((BPW * S,), jnp.int32),
            pltpu.VMEM((NBUF, CB * S, EMBED), jnp.float32),
            pltpu.VMEM((NBUF, CB * S, FEAT), jnp.float32),
            pltpu.VMEM((NBUF, CB * S, FEAT), jnp.float32),
            pltpu.VMEM((NBUF, CB * S, FEAT), jnp.float32),
            pltpu.VMEM((NBUF, CB, 128), jnp.float32),
            pltpu.VMEM((NBUF, CB, 128), jnp.float32),
            pltpu.VMEM((NBUF, CB, 128), jnp.float32),
            pltpu.SemaphoreType.DMA((NBUF,)),
            pltpu.SemaphoreType.DMA((NBUF,)),
            pltpu.SemaphoreType.DMA,
            pltpu.VMEM_SHARED((1500, EMBED), jnp.float32),
            pltpu.VMEM_SHARED((400, FEAT), jnp.float32),
            pltpu.VMEM_SHARED((600, FEAT), jnp.float32),
            pltpu.VMEM_SHARED((20, FEAT), jnp.float32),
        ],
    )
    def k(sp_ids_h, ab_ids_h, it_ids_h, te_ids_h,
          sp_th, ab_th, it_th, te_th,
          sp_o, abit_o, tep_o,
          i_sp, i_ab, i_it, i_te,
          d_sp, d_ab, d_it, d_te, o_sp, o_abit, o_tep,
          gsem, wsem, isem,
          sp_t, ab_t, it_t, te_t):
        sid = lax.axis_index("s")
        wid = sid * NC + lax.axis_index("c")
        w0 = wid * BPW

        @pl.when(sid == 0)
        def _():
            tcps = [
                pltpu.async_copy(sp_th, sp_t, isem),
                pltpu.async_copy(ab_th, ab_t, isem),
                pltpu.async_copy(it_th, it_t, isem),
                pltpu.async_copy(te_th, te_t, isem),
            ]
            for cp in tcps:
                cp.wait()

        idcps = [
            pltpu.async_copy(sp_ids_h.at[pl.ds(w0 * S, BPW * S)], i_sp, isem),
            pltpu.async_copy(ab_ids_h.at[pl.ds(w0 * S, BPW * S)], i_ab, isem),
            pltpu.async_copy(it_ids_h.at[pl.ds(w0 * S, BPW * S)], i_it, isem),
            pltpu.async_copy(te_ids_h.at[pl.ds(w0 * S, BPW * S)], i_te, isem),
        ]
        for cp in idcps:
            cp.wait()
        plsc.subcore_barrier()

        def fire_gather(c):
            b = c % NBUF
            o = c * CB * S
            return [
                pltpu.async_copy(sp_t.at[i_sp.at[pl.ds(o, CB * S)]],
                                 d_sp.at[b], gsem.at[b]),
                pltpu.async_copy(ab_t.at[i_ab.at[pl.ds(o, CB * S)]],
                                 d_ab.at[b], gsem.at[b]),
                pltpu.async_copy(it_t.at[i_it.at[pl.ds(o, CB * S)]],
                                 d_it.at[b], gsem.at[b]),
                pltpu.async_copy(te_t.at[i_te.at[pl.ds(o, CB * S)]],
                                 d_te.at[b], gsem.at[b]),
            ]

        def repack(c):
            b = c % NBUF
            dsp, dab = d_sp.at[b], d_ab.at[b]
            dit, dte = d_it.at[b], d_te.at[b]
            osp, oab, ote = o_sp.at[b], o_abit.at[b], o_tep.at[b]
            zero = jnp.zeros((16,), jnp.float32)

            def body(r, carry):
                for s in range(S):
                    p = r * S + s
                    osp[r, pl.ds(32 * s, 16)] = dsp[p, pl.ds(0, 16)]
                    osp[r, pl.ds(32 * s + 16, 16)] = dsp[p, pl.ds(16, 16)]
                    oab[r, pl.ds(16 * s, 16)] = dab[p]
                    oab[r, pl.ds(64 + 16 * s, 16)] = dit[p]
                    ote[r, pl.ds(16 * s, 16)] = dte[p]
                    ote[r, pl.ds(64 + 16 * s, 16)] = zero
                return carry

            lax.fori_loop(0, CB, body, 0)

        def fire_write(c):
            b = c % NBUF
            r0 = w0 + c * CB
            return [
                pltpu.async_copy(o_sp.at[b], sp_o.at[pl.ds(r0, CB)], wsem.at[b]),
                pltpu.async_copy(o_abit.at[b], abit_o.at[pl.ds(r0, CB)], wsem.at[b]),
                pltpu.async_copy(o_tep.at[b], tep_o.at[pl.ds(r0, CB)], wsem.at[b]),
            ]

        gcps = {c: fire_gather(c) for c in range(min(NBUF, NCH))}
        wcps = {}
        for c in range(NCH):
            for cp in gcps.pop(c):
                cp.wait()
            repack(c)
            wcps[c] = fire_write(c)
            nxt = c + NBUF
            if nxt < NCH:
                for cp in wcps.pop(c):
                    cp.wait()
                gcps[nxt] = fire_gather(nxt)
        for c, cps in wcps.items():
            for cp in cps:
                cp.wait()

    return k(sp1d, ab1d, it1d, te1d, sp_table, ab_table, it_table, te_table)


def _sc_gather_b(mv1d, mv_table):
    @functools.partial(
        pl.kernel,
        mesh=plsc.VectorSubcoreMesh(**_SC_MESH),
        compiler_params=pltpu.CompilerParams(use_tc_tiling_on_sc=False),
        out_type=jax.ShapeDtypeStruct((B, 128), jnp.float32),
        scratch_types=[
            pltpu.VMEM((BPW * S * M,), jnp.int32),
            pltpu.VMEM((NBUF, CB * S * M, FEAT), jnp.float32),
            pltpu.VMEM((NBUF, CB, 128), jnp.float32),
            pltpu.SemaphoreType.DMA((NBUF,)),
            pltpu.SemaphoreType.DMA((NBUF,)),
            pltpu.SemaphoreType.DMA,
            pltpu.VMEM_SHARED((1000, FEAT), jnp.float32),
        ],
    )
    def k(mv_ids_h, mv_th, mvp_o, i_mv, d_mv, o_mv, gsem, wsem, isem, mv_t):
        sid = lax.axis_index("s")
        wid = sid * NC + lax.axis_index("c")
        w0 = wid * BPW

        @pl.when(sid == 0)
        def _():
            pltpu.async_copy(mv_th, mv_t, isem).wait()

        pltpu.async_copy(
            mv_ids_h.at[pl.ds(w0 * S * M, BPW * S * M)], i_mv, isem).wait()
        plsc.subcore_barrier()

        def fire_gather(c):
            b = c % NBUF
            om = c * CB * S * M
            return [pltpu.async_copy(mv_t.at[i_mv.at[pl.ds(om, CB * S * M)]],
                                     d_mv.at[b], gsem.at[b])]

        def repack(c):
            b = c % NBUF
            dmv, omv = d_mv.at[b], o_mv.at[b]
            zero = jnp.zeros((16,), jnp.float32)

            def body(r, carry):
                for s in range(S):
                    p = (r * S + s) * M
                    acc = dmv[p] + dmv[p + 1] + dmv[p + 2] + dmv[p + 3]
                    omv[r, pl.ds(16 * s, 16)] = acc
                    omv[r, pl.ds(64 + 16 * s, 16)] = zero
                return carry

            lax.fori_loop(0, CB, body, 0)

        def fire_write(c):
            b = c % NBUF
            return [pltpu.async_copy(o_mv.at[b],
                                     mvp_o.at[pl.ds(w0 + c * CB, CB)],
                                     wsem.at[b])]

        gcps = {c: fire_gather(c) for c in range(min(NBUF, NCH))}
        wcps = {}
        for c in range(NCH):
            for cp in gcps.pop(c):
                cp.wait()
            repack(c)
            wcps[c] = fire_write(c)
            nxt = c + NBUF
            if nxt < NCH:
                for cp in wcps.pop(c):
                    cp.wait()
                gcps[nxt] = fire_gather(nxt)
        for c, cps in wcps.items():
            for cp in cps:
                cp.wait()

    return k(mv1d, mv_table)


def _mlp_body(sp_r, abit_r, tep_r, mvp_r, num_r,
              wsp_r, wabit_r, wtep_r, wmvp_r, bpe_r,
              w1f_r, w1n_r, b1_r, w2_r, b2_r, w3_r, b3_r,
              wv1_r, bv1_r, wv2_r, bv2_r,
              wp1x_r, wp1s_r, bp1_r, wp2_r, bp2_r,
              v_o, p0_o, p1_o):
    dot = functools.partial(jnp.dot, preferred_element_type=jnp.float32)
    bf = lambda x: x.astype(jnp.bfloat16)
    enc = dot(bf(sp_r[:]), wsp_r[:]) + dot(bf(abit_r[:]), wabit_r[:]) \
        + dot(bf(tep_r[:]), wtep_r[:]) + dot(bf(mvp_r[:]), wmvp_r[:]) + bpe_r[:]
    encb = bf(jnp.maximum(enc, 0.0))
    x = jnp.maximum(dot(encb, w1f_r[:]) + dot(bf(num_r[:]), w1n_r[:]) + b1_r[:],
                    0.0)
    x = bf(x)
    x = bf(jnp.maximum(dot(x, w2_r[:]) + b2_r[:], 0.0))
    x = bf(jnp.maximum(dot(x, w3_r[:]) + b3_r[:], 0.0))
    v = dot(bf(jnp.maximum(dot(x, wv1_r[:]) + bv1_r[:], 0.0)), wv2_r[:]) \
        + bv2_r[:]
    v_o[:] = v[:, 0]
    for i, p_o in ((0, p0_o), (1, p1_o)):
        slot = encb[:, i * POKE:(i + 1) * POKE]
        h = jnp.maximum(dot(x, wp1x_r[:]) + dot(slot, wp1s_r[:]) + bp1_r[:], 0.0)
        p_o[:] = dot(bf(h), wp2_r[:]) + bp2_r[:]


def kernel(species_ids, move_ids, ability_ids, item_ids, tera_ids, numeric, sp_table, mv_table, ab_table, it_table, te_table, W_pe, b_pe, W1, b1, g1, be1, W2, b2, g2, be2, W3, b3, g3, be3, Wv1, bv1, Wv2, bv2, Wp1, bp1, Wp2, bp2):
    f32 = jnp.float32
    sp2, abit, tep = _sc_gather_a(
        species_ids.reshape(-1), ability_ids.reshape(-1),
        item_ids.reshape(-1), tera_ids.reshape(-1),
        sp_table, ab_table, it_table, te_table)
    mv1d, sp2 = lax.optimization_barrier((move_ids.reshape(-1), sp2))
    mvp = _sc_gather_b(mv1d, mv_table)

    def bd16(blk_lo, blk_hi):
        z = jnp.zeros((128, S * POKE), f32)
        for s in range(S):
            z = z.at[16 * s:16 * (s + 1), POKE * s:POKE * (s + 1)].set(blk_lo)
            if blk_hi is not None:
                z = z.at[64 + 16 * s:64 + 16 * (s + 1),
                         POKE * s:POKE * (s + 1)].set(blk_hi)
        return z

    wsp = jnp.zeros((128, S * POKE), f32)
    for s in range(S):
        wsp = wsp.at[32 * s:32 * (s + 1), POKE * s:POKE * (s + 1)].set(W_pe[0:32])
    wabit = bd16(W_pe[48:64], W_pe[64:80])
    wtep = bd16(W_pe[80:96], None)
    wmvp = bd16(W_pe[32:48], None)
    bpe = jnp.tile(b_pe, S).reshape(1, S * POKE)

    inv = 1.0 / jnp.sqrt(1.0 + 1e-5)
    w1 = W1 * (g1 * inv)[None, :]
    b1f = (b1 * g1 * inv + be1).reshape(1, HID)
    w2 = W2 * (g2 * inv)[None, :]
    b2f = (b2 * g2 * inv + be2).reshape(1, HID)
    w3 = W3 * (g3 * inv)[None, :]
    b3f = (b3 * g3 * inv + be3).reshape(1, HID // 2)
    w1f, w1n = w1[:S * POKE], w1[S * POKE:]

    grid = (B // R,)
    bf16 = jnp.bfloat16
    row = lambda c: pl.BlockSpec((R, c), lambda i: (i, 0))
    full = lambda a: pl.BlockSpec(a.shape, lambda i: (0,) * a.ndim)
    wargs = (wsp.astype(bf16), wabit.astype(bf16), wtep.astype(bf16),
             wmvp.astype(bf16), bpe,
             w1f.astype(bf16), w1n.astype(bf16), b1f,
             w2.astype(bf16), b2f, w3.astype(bf16), b3f,
             Wv1.astype(bf16), bv1.reshape(1, 64), Wv2.astype(bf16),
             bv2.reshape(1, 1),
             Wp1[:HID // 2].astype(bf16), Wp1[HID // 2:].astype(bf16),
             bp1.reshape(1, HID // 2),
             Wp2.astype(bf16), bp2.reshape(1, NUM_ACTIONS))
    v, p0, p1 = pl.pallas_call(
        _mlp_body,
        grid=grid,
        in_specs=[row(128), row(128), row(128), row(128), row(NUMERIC)]
                 + [full(a) for a in wargs],
        out_specs=[pl.BlockSpec((R,), lambda i: (i,)),
                   row(NUM_ACTIONS), row(NUM_ACTIONS)],
        out_shape=[jax.ShapeDtypeStruct((B,), f32),
                   jax.ShapeDtypeStruct((B, NUM_ACTIONS), f32),
                   jax.ShapeDtypeStruct((B, NUM_ACTIONS), f32)],
        compiler_params=pltpu.CompilerParams(
            dimension_semantics=("parallel",)),
    )(sp2, abit, tep, mvp, numeric, *wargs)
    return (v, p0, p1)
